# baseline (speedup 1.0000x reference)
import functools

import jax
import jax.numpy as jnp
from jax.experimental import pallas as pl
from jax.experimental.pallas import tpu as pltpu
from jax.experimental.pallas import tpu_sc as plsc

B, N, C = 2, 1000, 256
PH, PW = 7, 7
NCELL = PH * PW
M = B * N * NCELL
M_PAD = 98304
K = 4 * M_PAD
GW = 128
BM = 256

LVL_W = (256, 128, 64, 32)
LVL_OFF = (0, 2 * 256 * 256, 2 * 256 * 256 + 2 * 128 * 128,
           2 * 256 * 256 + 2 * 128 * 128 + 2 * 64 * 64)


def _index_weight_body(rois_ref, lvl_ref, i00, i01, i10, i11,
                       w00, w01, w10, w11):
    r = rois_ref[...]
    y1 = r[:, 0:1]
    x1 = r[:, 1:2]
    y2 = r[:, 2:3]
    x2 = r[:, 3:4]
    lvl = lvl_ref[...]
    is2 = lvl == 2
    is3 = lvl == 3
    is4 = lvl == 4
    w = jnp.where(is2, LVL_W[0], jnp.where(is3, LVL_W[1],
                                           jnp.where(is4, LVL_W[2], LVL_W[3])))
    base = jnp.where(is2, LVL_OFF[0], jnp.where(is3, LVL_OFF[1],
                                                jnp.where(is4, LVL_OFF[2], LVL_OFF[3])))
    row = jax.lax.broadcasted_iota(jnp.int32, (B * N, 1), 0)
    base = base + jnp.where(row >= N, w * w, 0)
    hm1 = (w - 1).astype(jnp.float32)
    t = (jax.lax.broadcasted_iota(jnp.int32, (1, PH), 1).astype(jnp.float32)
         / float(PH - 1))
    ys = (y1 + (y2 - y1) * t) * hm1
    xs = (x1 + (x2 - x1) * t) * hm1
    y0 = jnp.clip(jnp.floor(ys), 0.0, hm1 - 1.0)
    x0 = jnp.clip(jnp.floor(xs), 0.0, hm1 - 1.0)
    wy = jnp.clip(ys - y0, 0.0, 1.0)
    wx = jnp.clip(xs - x0, 0.0, 1.0)
    y0i = y0.astype(jnp.int32)
    x0i = x0.astype(jnp.int32)

    cell = jax.lax.broadcasted_iota(jnp.int32, (1, NCELL), 1)
    pyv = cell // PW
    pxv = cell - pyv * PW

    def expand(src, sel):
        acc = jnp.zeros((B * N, NCELL), src.dtype)
        for p in range(PH):
            acc = jnp.where(sel == p, src[:, p:p + 1], acc)
        return acc

    y0c = expand(y0i, pyv)
    x0c = expand(x0i, pxv)
    wyc = expand(wy, pyv)
    wxc = expand(wx, pxv)
    f00 = base + y0c * w + x0c
    i00[...] = f00
    i01[...] = f00 + 1
    i10[...] = f00 + w
    i11[...] = f00 + w + 1
    wy1 = 1.0 - wyc
    wx1 = 1.0 - wxc
    w00[...] = wy1 * wx1
    w01[...] = wy1 * wxc
    w10[...] = wyc * wx1
    w11[...] = wyc * wxc


def _compute_index_weights(rois2d, lvl, interpret=False):
    outs = ([jax.ShapeDtypeStruct((B * N, NCELL), jnp.int32)] * 4
            + [jax.ShapeDtypeStruct((B * N, NCELL), jnp.float32)] * 4)
    return pl.pallas_call(_index_weight_body, out_shape=outs,
                          interpret=interpret)(rois2d, lvl)


def _sc_gather(table, idx):
    cw = table.shape[1]
    mesh = plsc.VectorSubcoreMesh(core_axis_name="c", subcore_axis_name="s")

    @functools.partial(pl.kernel, mesh=mesh,
                       out_type=jax.ShapeDtypeStruct((K, cw), table.dtype))
    def k(table_hbm, idx_hbm, out_hbm):
        def body(i_vmem, o_vmem):
            pltpu.sync_copy(table_hbm.at[i_vmem.at[0]], o_vmem)

        pltpu.emit_pipeline(
            body,
            grid=(K // GW,),
            in_specs=[pl.BlockSpec((1, GW), lambda i: (0, i))],
            out_specs=[pl.BlockSpec((GW, cw), lambda i: (i, 0))],
            core_axis_name=("c", "s"),
            dimension_semantics=(pltpu.PARALLEL,),
        )(idx_hbm, out_hbm)

    return k(table, idx)


def _blend_body(w0, w1, w2, w3, g0, g1, g2, g3, o):
    o[...] = (w0[...] * g0[...].astype(jnp.float32)
              + w1[...] * g1[...].astype(jnp.float32)
              + w2[...] * g2[...].astype(jnp.float32)
              + w3[...] * g3[...].astype(jnp.float32))


def _blend(gathered, w00, w01, w10, w11, interpret=False):
    nblk = (M + BM - 1) // BM
    gstride = M_PAD // BM

    def gspec(j):
        return pl.BlockSpec((BM, C), lambda i, j=j: (j * gstride + i, 0))

    wspec = pl.BlockSpec((BM, 1), lambda i: (i, 0))
    return pl.pallas_call(
        _blend_body,
        grid=(nblk,),
        in_specs=[wspec] * 4 + [gspec(0), gspec(1), gspec(2), gspec(3)],
        out_specs=pl.BlockSpec((BM, C), lambda i: (i, 0)),
        out_shape=jax.ShapeDtypeStruct((M, C), jnp.float32),
        interpret=interpret,
    )(w00, w01, w10, w11, gathered, gathered, gathered, gathered)


def _roi_levels(rois, image_meta):
    boxes = jax.lax.stop_gradient(rois)
    h = boxes[..., 2] - boxes[..., 0]
    w = boxes[..., 3] - boxes[..., 1]
    img_area = image_meta[:, 4] * image_meta[:, 5]
    spec = jnp.log2(jnp.sqrt(jnp.maximum(h * w, 1e-12))
                    / (224.0 / jnp.sqrt(img_area))[:, None])
    return jnp.minimum(5, jnp.maximum(2, 4 + jnp.round(spec).astype(jnp.int32)))


def kernel(rois, image_meta, feat_p2, feat_p3, feat_p4, feat_p5):
    rois2d = rois.reshape(B * N, 4)
    lvl = _roi_levels(rois, image_meta).reshape(B * N, 1)
    i00, i01, i10, i11, w00, w01, w10, w11 = _compute_index_weights(rois2d, lvl)
    table = jnp.concatenate(
        [feat_p2.astype(jnp.bfloat16).reshape(-1, C),
         feat_p3.astype(jnp.bfloat16).reshape(-1, C),
         feat_p4.astype(jnp.bfloat16).reshape(-1, C),
         feat_p5.astype(jnp.bfloat16).reshape(-1, C)], axis=0)
    table = jax.lax.bitcast_convert_type(
        table.reshape(table.shape[0], C // 2, 2), jnp.int32)
    pad = jnp.zeros((M_PAD - M,), jnp.int32)
    idx = jnp.concatenate(
        [i00.reshape(-1), pad, i01.reshape(-1), pad,
         i10.reshape(-1), pad, i11.reshape(-1), pad]).reshape(1, K)
    gathered = _sc_gather(table, idx)
    gathered = jax.lax.bitcast_convert_type(gathered, jnp.bfloat16).reshape(K, C)
    out = _blend(gathered,
                 w00.reshape(M, 1), w01.reshape(M, 1),
                 w10.reshape(M, 1), w11.reshape(M, 1))
    return out.reshape(B, N, PH, PW, C)


# speedup vs baseline: 3.0445x; 3.0445x over previous
import functools

import jax
import jax.numpy as jnp
from jax.experimental import pallas as pl
from jax.experimental.pallas import tpu as pltpu
from jax.experimental.pallas import tpu_sc as plsc

B, N, C = 2, 1000, 256
PH, PW = 7, 7
NCELL = PH * PW
M = B * N * NCELL
M_PAD = 98304
K = 4 * M_PAD
GW = 128
BM = 256

LVL_W = (256, 128, 64, 32)
LVL_OFF = (0, 2 * 256 * 256, 2 * 256 * 256 + 2 * 128 * 128,
           2 * 256 * 256 + 2 * 128 * 128 + 2 * 64 * 64)
TBL_ROWS = LVL_OFF[3] + 2 * 32 * 32


def _index_weight_body(rois_ref, lvl_ref, i00, i01, i10, i11,
                       w00, w01, w10, w11):
    r = rois_ref[...]
    y1 = r[:, 0:1]
    x1 = r[:, 1:2]
    y2 = r[:, 2:3]
    x2 = r[:, 3:4]
    lvl = lvl_ref[...]
    is2 = lvl == 2
    is3 = lvl == 3
    is4 = lvl == 4
    w = jnp.where(is2, LVL_W[0], jnp.where(is3, LVL_W[1],
                                           jnp.where(is4, LVL_W[2], LVL_W[3])))
    base = jnp.where(is2, LVL_OFF[0], jnp.where(is3, LVL_OFF[1],
                                                jnp.where(is4, LVL_OFF[2], LVL_OFF[3])))
    row = jax.lax.broadcasted_iota(jnp.int32, (B * N, 1), 0)
    base = base + jnp.where(row >= N, w * w, 0)
    hm1 = (w - 1).astype(jnp.float32)
    t = (jax.lax.broadcasted_iota(jnp.int32, (1, PH), 1).astype(jnp.float32)
         / float(PH - 1))
    ys = (y1 + (y2 - y1) * t) * hm1
    xs = (x1 + (x2 - x1) * t) * hm1
    y0 = jnp.clip(jnp.floor(ys), 0.0, hm1 - 1.0)
    x0 = jnp.clip(jnp.floor(xs), 0.0, hm1 - 1.0)
    wy = jnp.clip(ys - y0, 0.0, 1.0)
    wx = jnp.clip(xs - x0, 0.0, 1.0)
    y0i = y0.astype(jnp.int32)
    x0i = x0.astype(jnp.int32)

    cell = jax.lax.broadcasted_iota(jnp.int32, (1, NCELL), 1)
    pyv = cell // PW
    pxv = cell - pyv * PW

    def expand(src, sel):
        acc = jnp.zeros((B * N, NCELL), src.dtype)
        for p in range(PH):
            acc = jnp.where(sel == p, src[:, p:p + 1], acc)
        return acc

    y0c = expand(y0i, pyv)
    x0c = expand(x0i, pxv)
    wyc = expand(wy, pyv)
    wxc = expand(wx, pxv)
    f00 = base + y0c * w + x0c
    i00[...] = f00
    i01[...] = f00 + 1
    i10[...] = f00 + w
    i11[...] = f00 + w + 1
    wy1 = 1.0 - wyc
    wx1 = 1.0 - wxc
    w00[...] = wy1 * wx1
    w01[...] = wy1 * wxc
    w10[...] = wyc * wx1
    w11[...] = wyc * wxc


def _compute_index_weights(rois2d, lvl, interpret=False):
    outs = ([jax.ShapeDtypeStruct((B * N, NCELL), jnp.int32)] * 4
            + [jax.ShapeDtypeStruct((B * N, NCELL), jnp.float32)] * 4)
    return pl.pallas_call(_index_weight_body, out_shape=outs,
                          interpret=interpret)(rois2d, lvl)


def _bf16_bits(x_i32):
    return (x_i32 + 0x7fff + ((x_i32 >> 16) & 1)) & jnp.int32(-65536)


def _pack_body(x_ref, _t_ref, t_out):
    x = x_ref[...]
    b_lo = jax.lax.bitcast_convert_type(x[:, :C // 2], jnp.int32)
    b_hi = jax.lax.bitcast_convert_type(x[:, C // 2:], jnp.int32)
    lo = jax.lax.shift_right_logical(_bf16_bits(b_lo), 16)
    t_out[...] = lo | _bf16_bits(b_hi)


def _pack_first_body(x_ref, t_out):
    _pack_body(x_ref, None, t_out)


def _pack_level(feat2d, tbl, row_off, interpret=False):
    rows = feat2d.shape[0]
    br = min(rows, 2048)
    out_spec = pl.BlockSpec((br, C // 2), lambda i, o=row_off // br: (o + i, 0))
    out_shape = jax.ShapeDtypeStruct((TBL_ROWS, C // 2), jnp.int32)
    in_spec = pl.BlockSpec((br, C), lambda i: (i, 0))
    if tbl is None:
        return pl.pallas_call(
            _pack_first_body, grid=(rows // br,), in_specs=[in_spec],
            out_specs=out_spec, out_shape=out_shape, interpret=interpret,
        )(feat2d)
    return pl.pallas_call(
        _pack_body, grid=(rows // br,),
        in_specs=[in_spec, pl.BlockSpec(memory_space=pl.ANY)],
        out_specs=out_spec, out_shape=out_shape,
        input_output_aliases={1: 0}, interpret=interpret,
    )(feat2d, tbl)


def _sc_gather(table, idx):
    cw = table.shape[1]
    mesh = plsc.VectorSubcoreMesh(core_axis_name="c", subcore_axis_name="s")

    @functools.partial(pl.kernel, mesh=mesh,
                       out_type=jax.ShapeDtypeStruct((K, cw), table.dtype))
    def k(table_hbm, idx_hbm, out_hbm):
        def body(i_vmem, o_vmem):
            pltpu.sync_copy(table_hbm.at[i_vmem.at[0]], o_vmem)

        pltpu.emit_pipeline(
            body,
            grid=(K // GW,),
            in_specs=[pl.BlockSpec((1, GW), lambda i: (0, i))],
            out_specs=[pl.BlockSpec((GW, cw), lambda i: (i, 0))],
            core_axis_name=("c", "s"),
            dimension_semantics=(pltpu.PARALLEL,),
        )(idx_hbm, out_hbm)

    return k(table, idx)


def _blend_body(w0, w1, w2, w3, g0, g1, g2, g3, o):
    acc_lo = None
    acc_hi = None
    for w_ref, g_ref in ((w0, g0), (w1, g1), (w2, g2), (w3, g3)):
        w = w_ref[...]
        g = g_ref[...]
        lo = jax.lax.bitcast_convert_type(g << 16, jnp.float32)
        hi = jax.lax.bitcast_convert_type(g & jnp.int32(-65536), jnp.float32)
        acc_lo = w * lo if acc_lo is None else acc_lo + w * lo
        acc_hi = w * hi if acc_hi is None else acc_hi + w * hi
    o[:, :C // 2] = acc_lo
    o[:, C // 2:] = acc_hi


def _blend(gathered, w00, w01, w10, w11, interpret=False):
    nblk = (M + BM - 1) // BM
    gstride = M_PAD // BM

    def gspec(j):
        return pl.BlockSpec((BM, C // 2), lambda i, j=j: (j * gstride + i, 0))

    wspec = pl.BlockSpec((BM, 1), lambda i: (i, 0))
    return pl.pallas_call(
        _blend_body,
        grid=(nblk,),
        in_specs=[wspec] * 4 + [gspec(0), gspec(1), gspec(2), gspec(3)],
        out_specs=pl.BlockSpec((BM, C), lambda i: (i, 0)),
        out_shape=jax.ShapeDtypeStruct((M, C), jnp.float32),
        interpret=interpret,
    )(w00, w01, w10, w11, gathered, gathered, gathered, gathered)


def _roi_levels(rois, image_meta):
    boxes = jax.lax.stop_gradient(rois)
    h = boxes[..., 2] - boxes[..., 0]
    w = boxes[..., 3] - boxes[..., 1]
    img_area = image_meta[:, 4] * image_meta[:, 5]
    spec = jnp.log2(jnp.sqrt(jnp.maximum(h * w, 1e-12))
                    / (224.0 / jnp.sqrt(img_area))[:, None])
    return jnp.minimum(5, jnp.maximum(2, 4 + jnp.round(spec).astype(jnp.int32)))


def kernel(rois, image_meta, feat_p2, feat_p3, feat_p4, feat_p5):
    rois2d = rois.reshape(B * N, 4)
    lvl = _roi_levels(rois, image_meta).reshape(B * N, 1)
    i00, i01, i10, i11, w00, w01, w10, w11 = _compute_index_weights(rois2d, lvl)
    table = None
    off = 0
    for feat in (feat_p2, feat_p3, feat_p4, feat_p5):
        f2d = feat.reshape(-1, C)
        table = _pack_level(f2d, table, off)
        off += f2d.shape[0]
    pad = jnp.zeros((M_PAD - M,), jnp.int32)
    idx = jnp.concatenate(
        [i00.reshape(-1), pad, i01.reshape(-1), pad,
         i10.reshape(-1), pad, i11.reshape(-1), pad]).reshape(1, K)
    gathered = _sc_gather(table, idx)
    out = _blend(gathered,
                 w00.reshape(M, 1), w01.reshape(M, 1),
                 w10.reshape(M, 1), w11.reshape(M, 1))
    return out.reshape(B, N, PH, PW, C)
